# baseline (speedup 1.0000x reference)
import jax
import jax.numpy as jnp
from jax import lax
from jax.experimental import pallas as pl
from jax.experimental.pallas import tpu as pltpu
from jax.experimental.pallas import tpu_sc as plsc

B = 16384
K = 50
D = 32
TOK = 1000000
NC, NS, L = 2, 16, 16
DH = D // NC
TPS = B // NS
NCHK = 8
CHS = TPS // NCHK
ENT = K * CHS
NSTG = 8
STG = TOK // NSTG

_mesh = plsc.VectorSubcoreMesh(core_axis_name="c", subcore_axis_name="s")


def _body(xp_hbm, w_hbm, out_hbm, idx_t, g0, g1, acc, spm, sem0, sem1):
    cid = lax.axis_index("c")
    sid = lax.axis_index("s")

    pltpu.sync_copy(xp_hbm.at[pl.ds(sid * (TPS * K), TPS * K)], idx_t)

    def start(ch, buf, sem):
        pltpu.async_copy(spm.at[idx_t.at[pl.ds(ch * ENT, ENT)]], buf, sem)

    def wait(buf, sem):
        pltpu.make_async_copy(spm.at[idx_t.at[pl.ds(0, ENT)]], buf, sem).wait()

    def reduce_chunk(buf, ch):
        accs = tuple(buf[pl.ds(g * L, L)] for g in range(CHS // L))

        def jbody(j, a):
            return tuple(a[g] + buf[pl.ds(j * CHS + g * L, L)]
                         for g in range(CHS // L))

        accs = lax.fori_loop(1, K, jbody, accs)
        for g in range(CHS // L):
            acc[pl.ds(ch * CHS + g * L, L)] = accs[g]

    def per_dim(d, carry):
        dg = cid * DH + d

        @pl.when(sid < NSTG)
        def _stage():
            pltpu.sync_copy(w_hbm.at[pl.ds(dg * TOK + sid * STG, STG)],
                            spm.at[pl.ds(sid * STG, STG)])

        plsc.subcore_barrier()
        start(0, g0, sem0)
        start(1, g1, sem1)
        for ch in range(NCHK):
            buf, sem = (g0, sem0) if ch % 2 == 0 else (g1, sem1)
            wait(buf, sem)
            reduce_chunk(buf, ch)
            if ch + 2 < NCHK:
                start(ch + 2, buf, sem)
        pltpu.sync_copy(acc, out_hbm.at[cid, d, pl.ds(sid * TPS, TPS)])
        plsc.subcore_barrier()
        return carry

    lax.fori_loop(0, DH, per_dim, 0)


_emb_sum = pl.kernel(
    _body,
    out_type=jax.ShapeDtypeStruct((NC, DH, B), jnp.float32),
    mesh=_mesh,
    scratch_types=[
        pltpu.VMEM((TPS * K,), jnp.int32),
        pltpu.VMEM((ENT,), jnp.float32),
        pltpu.VMEM((ENT,), jnp.float32),
        pltpu.VMEM((TPS,), jnp.float32),
        pltpu.VMEM_SHARED((TOK,), jnp.float32),
        pltpu.SemaphoreType.DMA,
        pltpu.SemaphoreType.DMA,
    ],
    compiler_params=pltpu.CompilerParams(use_tc_tiling_on_sc=False),
)


def kernel(x, weight):
    wflat = weight.T.reshape(D * TOK)
    xp = x.reshape(NS, NCHK, CHS, K).transpose(0, 1, 3, 2).reshape(-1)
    res = _emb_sum(xp, wflat)
    return res.reshape(D, B).T
